# baseline (speedup 1.0000x reference)
import jax
import jax.numpy as jnp
from jax.experimental import pallas as pl
from jax.experimental.pallas import tpu as pltpu

_SMOOTH = 1e-05
_K_FRAC = 10.0 / 100.0
_INT_MAX = 2**31 - 1


def _key_of(x):
    ki = jax.lax.bitcast_convert_type(x, jnp.int32)
    return jnp.where(ki < 0, ki ^ jnp.int32(0x7FFFFFFF), ki)


def _count_le(plane_i16, mid_i32):
    return jnp.sum(plane_i16 <= mid_i32.astype(jnp.int16), dtype=jnp.int32)


def _search16(plane_refs, ks, B, lo0, hi0):
    def body(_, carry):
        los, his, cbls = carry
        nlo, nhi, ncb = [], [], []
        for s in range(B):
            lo, hi, cbl = los[s], his[s], cbls[s]
            mid = (lo + hi) >> 1
            c = _count_le(plane_refs(s), mid)
            pred = c >= ks[s]
            nlo.append(jnp.where(pred, lo, mid + 1))
            nhi.append(jnp.where(pred, mid, hi))
            ncb.append(jnp.where(pred, cbl, c))
        return (tuple(nlo), tuple(nhi), tuple(ncb))

    init = (
        tuple(jnp.int32(lo0) for _ in range(B)),
        tuple(jnp.int32(hi0) for _ in range(B)),
        tuple(jnp.int32(0) for _ in range(B)),
    )
    _, thrs, cbls = jax.lax.fori_loop(0, 16, body, init)
    return thrs, cbls


def _dice_body(l0_ref, l1_ref, tgt_ref, out_ref, mkey_ref, h16_ref):
    B = l0_ref.shape[0]

    ks = []
    for s in range(B):
        x = l1_ref[s] - l0_ref[s]
        fg = tgt_ref[s] == 1
        mkey = jnp.where(fg, _key_of(x), jnp.int32(_INT_MAX))
        mkey_ref[s] = mkey
        h16_ref[s] = (mkey >> 16).astype(jnp.int16)
        n = jnp.sum(fg.astype(jnp.int32))
        ks.append(jnp.maximum(
            jnp.int32(1),
            jnp.floor(n.astype(jnp.float32) * jnp.float32(_K_FRAC)).astype(jnp.int32),
        ))

    buckets, cbls = _search16(lambda s: h16_ref[s], ks, B, -(2**15), 2**15 - 1)

    for s in range(B):
        mkey = mkey_ref[s]
        hk = mkey >> 16
        low = (mkey & jnp.int32(0xFFFF)) - jnp.int32(32768)
        key2 = jnp.where(
            hk == buckets[s], low,
            jnp.where(hk < buckets[s], jnp.int32(-32768), jnp.int32(32767)),
        )
        h16_ref[s] = key2.astype(jnp.int16)

    lows, _ = _search16(lambda s: h16_ref[s], ks, B, -(2**15), 2**15 - 1)

    acc = jnp.float32(0.0)
    for s in range(B):
        thr = (buckets[s] << 16) | (lows[s] + jnp.int32(32768))
        x = l1_ref[s] - l0_ref[s]
        fg = tgt_ref[s] == 1
        key = _key_of(x)
        p = jax.nn.sigmoid(x)
        ign = jnp.logical_and(fg, key > thr)
        s_all = jnp.sum(p)
        s_fg = jnp.sum(jnp.where(fg, p, jnp.float32(0.0)))
        s_ign = jnp.sum(jnp.where(ign, p, jnp.float32(0.0)))
        n = jnp.sum(fg.astype(jnp.int32))
        c_ign = jnp.sum(ign.astype(jnp.int32))
        inter = s_fg - s_ign
        union = (s_all - s_ign) + (n - c_ign).astype(jnp.float32)
        acc += (2.0 * inter + jnp.float32(_SMOOTH)) / (union + jnp.float32(_SMOOTH))

    out_ref[0] = jnp.float32(1.0) - acc / jnp.float32(B)


def kernel(logits, target):
    logits = logits.astype(jnp.float32)
    B, _, H, W = logits.shape
    R = (H * W) // 512
    l0 = logits[:, 0].reshape(B, R, 512)
    l1 = logits[:, 1].reshape(B, R, 512)
    tgt = target.astype(jnp.int32).reshape(B, R, 512)

    out = pl.pallas_call(
        _dice_body,
        in_specs=[
            pl.BlockSpec(l0.shape, lambda: (0, 0, 0)),
            pl.BlockSpec(l1.shape, lambda: (0, 0, 0)),
            pl.BlockSpec(tgt.shape, lambda: (0, 0, 0)),
        ],
        out_specs=pl.BlockSpec(memory_space=pltpu.SMEM),
        out_shape=jax.ShapeDtypeStruct((1,), jnp.float32),
        scratch_shapes=[
            pltpu.VMEM((B, R, 512), jnp.int32),
            pltpu.VMEM((B, R, 512), jnp.int16),
        ],
    )(l0, l1, tgt)
    return out[0]


# speedup vs baseline: 1.4730x; 1.4730x over previous
import jax
import jax.numpy as jnp
from jax.experimental import pallas as pl
from jax.experimental.pallas import tpu as pltpu

_SMOOTH = 1e-05
_K_FRAC = 10.0 / 100.0
_INT_MAX = 2**31 - 1


def _key_of(x):
    ki = jax.lax.bitcast_convert_type(x, jnp.int32)
    return jnp.where(ki < 0, ki ^ jnp.int32(0x7FFFFFFF), ki)


def _count_le(plane_i16, mid_i32):
    t = (plane_i16 <= mid_i32.astype(jnp.int16)).astype(jnp.int16)
    t = pltpu.bitcast(t, jnp.int32)
    while t.shape[0] > 16:
        h = t.shape[0] // 2
        t = t[:h] + t[h:]
    lo = jnp.bitwise_and(t, jnp.int32(0xFFFF))
    hi = jnp.right_shift(t, 16)
    return jnp.sum(lo + hi)


def _search16(plane_refs, ks, B, lo0, hi0):
    def body(_, carry):
        los, his, cbls = carry
        nlo, nhi, ncb = [], [], []
        for s in range(B):
            lo, hi, cbl = los[s], his[s], cbls[s]
            mid = (lo + hi) >> 1
            c = _count_le(plane_refs(s), mid)
            pred = c >= ks[s]
            nlo.append(jnp.where(pred, lo, mid + 1))
            nhi.append(jnp.where(pred, mid, hi))
            ncb.append(jnp.where(pred, cbl, c))
        return (tuple(nlo), tuple(nhi), tuple(ncb))

    init = (
        tuple(jnp.int32(lo0) for _ in range(B)),
        tuple(jnp.int32(hi0) for _ in range(B)),
        tuple(jnp.int32(0) for _ in range(B)),
    )
    _, thrs, cbls = jax.lax.fori_loop(0, 16, body, init)
    return thrs, cbls


def _dice_body(l0_ref, l1_ref, tgt_ref, out_ref, mkey_ref, h16_ref):
    B = l0_ref.shape[0]

    ks = []
    for s in range(B):
        x = l1_ref[s] - l0_ref[s]
        fg = tgt_ref[s] == 1
        mkey = jnp.where(fg, _key_of(x), jnp.int32(_INT_MAX))
        mkey_ref[s] = mkey
        h16_ref[s] = (mkey >> 16).astype(jnp.int16)
        n = jnp.sum(fg.astype(jnp.int32))
        ks.append(jnp.maximum(
            jnp.int32(1),
            jnp.floor(n.astype(jnp.float32) * jnp.float32(_K_FRAC)).astype(jnp.int32),
        ))

    buckets, cbls = _search16(lambda s: h16_ref[s], ks, B, -(2**15), 2**15 - 1)

    for s in range(B):
        mkey = mkey_ref[s]
        hk = mkey >> 16
        low = (mkey & jnp.int32(0xFFFF)) - jnp.int32(32768)
        key2 = jnp.where(
            hk == buckets[s], low,
            jnp.where(hk < buckets[s], jnp.int32(-32768), jnp.int32(32767)),
        )
        h16_ref[s] = key2.astype(jnp.int16)

    lows, _ = _search16(lambda s: h16_ref[s], ks, B, -(2**15), 2**15 - 1)

    acc = jnp.float32(0.0)
    for s in range(B):
        thr = (buckets[s] << 16) | (lows[s] + jnp.int32(32768))
        thr_f = jax.lax.bitcast_convert_type(
            jnp.where(thr < 0, thr ^ jnp.int32(0x7FFFFFFF), thr), jnp.float32)
        x = l1_ref[s] - l0_ref[s]
        fg = tgt_ref[s] == 1
        p = jax.nn.sigmoid(x)
        le = x <= thr_f
        keep = jnp.logical_and(fg, le)
        s_all = jnp.sum(p)
        pf = jnp.where(fg, p, jnp.float32(0.0))
        s_fg = jnp.sum(pf)
        s_keep = jnp.sum(jnp.where(le, pf, jnp.float32(0.0)))
        c_keep = jnp.sum(keep.astype(jnp.int32))
        union = s_all - s_fg + s_keep + c_keep.astype(jnp.float32)
        acc += ((2.0 * s_keep + jnp.float32(_SMOOTH))
                / (union + jnp.float32(_SMOOTH)))

    out_ref[0] = jnp.float32(1.0) - acc / jnp.float32(B)


def kernel(logits, target):
    logits = logits.astype(jnp.float32)
    B, _, H, W = logits.shape
    R = (H * W) // 512
    l0 = logits[:, 0].reshape(B, R, 512)
    l1 = logits[:, 1].reshape(B, R, 512)
    tgt = target.astype(jnp.int32).reshape(B, R, 512)

    out = pl.pallas_call(
        _dice_body,
        in_specs=[
            pl.BlockSpec(l0.shape, lambda: (0, 0, 0)),
            pl.BlockSpec(l1.shape, lambda: (0, 0, 0)),
            pl.BlockSpec(tgt.shape, lambda: (0, 0, 0)),
        ],
        out_specs=pl.BlockSpec(memory_space=pltpu.SMEM),
        out_shape=jax.ShapeDtypeStruct((1,), jnp.float32),
        scratch_shapes=[
            pltpu.VMEM((B, R, 512), jnp.int32),
            pltpu.VMEM((B, R, 512), jnp.int16),
        ],
    )(l0, l1, tgt)
    return out[0]


# speedup vs baseline: 1.6148x; 1.0962x over previous
import jax
import jax.numpy as jnp
from jax.experimental import pallas as pl
from jax.experimental.pallas import tpu as pltpu

_SMOOTH = 1e-05
_K_FRAC = 10.0 / 100.0
_INT_MAX = 2**31 - 1


def _key_of(x):
    ki = jax.lax.bitcast_convert_type(x, jnp.int32)
    return jnp.where(ki < 0, ki ^ jnp.int32(0x7FFFFFFF), ki)


def _count_le(plane_i16, mid_i32):
    t = (plane_i16 <= mid_i32.astype(jnp.int16)).astype(jnp.int16)
    while t.shape[0] > 16:
        h = t.shape[0] // 2
        t = t[:h] + t[h:]
    return jnp.sum(t.astype(jnp.int32))


def _search16(plane_refs, ks, B, lo0, hi0):
    def body(_, carry):
        los, his, cbls = carry
        nlo, nhi, ncb = [], [], []
        for s in range(B):
            lo, hi, cbl = los[s], his[s], cbls[s]
            mid = (lo + hi) >> 1
            c = _count_le(plane_refs(s), mid)
            pred = c >= ks[s]
            nlo.append(jnp.where(pred, lo, mid + 1))
            nhi.append(jnp.where(pred, mid, hi))
            ncb.append(jnp.where(pred, cbl, c))
        return (tuple(nlo), tuple(nhi), tuple(ncb))

    init = (
        tuple(jnp.int32(lo0) for _ in range(B)),
        tuple(jnp.int32(hi0) for _ in range(B)),
        tuple(jnp.int32(0) for _ in range(B)),
    )
    _, thrs, cbls = jax.lax.fori_loop(0, 16, body, init)
    return thrs, cbls


def _dice_body(l0_ref, l1_ref, tgt_ref, out_ref, mkey_ref, h16_ref):
    B = l0_ref.shape[0]

    ks = []
    for s in range(B):
        x = l1_ref[s] - l0_ref[s]
        fg = tgt_ref[s] == 1
        mkey = jnp.where(fg, _key_of(x), jnp.int32(_INT_MAX))
        mkey_ref[s] = mkey
        h16_ref[s] = (mkey >> 16).astype(jnp.int16)
        n = jnp.sum(fg.astype(jnp.int32))
        ks.append(jnp.maximum(
            jnp.int32(1),
            jnp.floor(n.astype(jnp.float32) * jnp.float32(_K_FRAC)).astype(jnp.int32),
        ))

    buckets, cbls = _search16(lambda s: h16_ref[s], ks, B, -(2**15), 2**15 - 1)

    for s in range(B):
        mkey = mkey_ref[s]
        hk = mkey >> 16
        low = (mkey & jnp.int32(0xFFFF)) - jnp.int32(32768)
        key2 = jnp.where(
            hk == buckets[s], low,
            jnp.where(hk < buckets[s], jnp.int32(-32768), jnp.int32(32767)),
        )
        h16_ref[s] = key2.astype(jnp.int16)

    lows, _ = _search16(lambda s: h16_ref[s], ks, B, -(2**15), 2**15 - 1)

    acc = jnp.float32(0.0)
    for s in range(B):
        thr = (buckets[s] << 16) | (lows[s] + jnp.int32(32768))
        thr_f = jax.lax.bitcast_convert_type(
            jnp.where(thr < 0, thr ^ jnp.int32(0x7FFFFFFF), thr), jnp.float32)
        x = l1_ref[s] - l0_ref[s]
        fg = tgt_ref[s] == 1
        p = jax.nn.sigmoid(x)
        le = x <= thr_f
        keep = jnp.logical_and(fg, le)
        s_all = jnp.sum(p)
        pf = jnp.where(fg, p, jnp.float32(0.0))
        s_fg = jnp.sum(pf)
        s_keep = jnp.sum(jnp.where(le, pf, jnp.float32(0.0)))
        c_keep = jnp.sum(keep.astype(jnp.int32))
        union = s_all - s_fg + s_keep + c_keep.astype(jnp.float32)
        acc += ((2.0 * s_keep + jnp.float32(_SMOOTH))
                / (union + jnp.float32(_SMOOTH)))

    out_ref[0] = jnp.float32(1.0) - acc / jnp.float32(B)


def kernel(logits, target):
    logits = logits.astype(jnp.float32)
    B, _, H, W = logits.shape
    R = (H * W) // 512
    l0 = logits[:, 0].reshape(B, R, 512)
    l1 = logits[:, 1].reshape(B, R, 512)
    tgt = target.astype(jnp.int32).reshape(B, R, 512)

    out = pl.pallas_call(
        _dice_body,
        in_specs=[
            pl.BlockSpec(l0.shape, lambda: (0, 0, 0)),
            pl.BlockSpec(l1.shape, lambda: (0, 0, 0)),
            pl.BlockSpec(tgt.shape, lambda: (0, 0, 0)),
        ],
        out_specs=pl.BlockSpec(memory_space=pltpu.SMEM),
        out_shape=jax.ShapeDtypeStruct((1,), jnp.float32),
        scratch_shapes=[
            pltpu.VMEM((B, R, 512), jnp.int32),
            pltpu.VMEM((B, R, 512), jnp.int16),
        ],
    )(l0, l1, tgt)
    return out[0]


# speedup vs baseline: 1.6429x; 1.0174x over previous
import jax
import jax.numpy as jnp
from jax.experimental import pallas as pl
from jax.experimental.pallas import tpu as pltpu

_SMOOTH = 1e-05
_K_FRAC = 10.0 / 100.0
_INT_MAX = 2**31 - 1


def _key_of(x):
    ki = jax.lax.bitcast_convert_type(x, jnp.int32)
    return jnp.where(ki < 0, ki ^ jnp.int32(0x7FFFFFFF), ki)


def _count_le(plane_i16, mid_i32):
    t = (plane_i16 <= mid_i32.astype(jnp.int16)).astype(jnp.int16)
    while t.shape[0] > 16:
        h = t.shape[0] // 2
        t = t[:h] + t[h:]
    return jnp.sum(t.astype(jnp.int32))


def _search16(plane_refs, ks, B, lo0, hi0):
    def body(_, carry):
        los, his, cbls = carry
        nlo, nhi, ncb = [], [], []
        for s in range(B):
            lo, hi, cbl = los[s], his[s], cbls[s]
            mid = (lo + hi) >> 1
            c = _count_le(plane_refs(s), mid)
            pred = c >= ks[s]
            nlo.append(jnp.where(pred, lo, mid + 1))
            nhi.append(jnp.where(pred, mid, hi))
            ncb.append(jnp.where(pred, cbl, c))
        return (tuple(nlo), tuple(nhi), tuple(ncb))

    init = (
        tuple(jnp.int32(lo0) for _ in range(B)),
        tuple(jnp.int32(hi0) for _ in range(B)),
        tuple(jnp.int32(0) for _ in range(B)),
    )
    _, thrs, cbls = jax.lax.fori_loop(0, 16, body, init, unroll=2)
    return thrs, cbls


def _dice_body(l0_ref, l1_ref, tgt_ref, out_ref, mkey_ref, h16_ref):
    B = l0_ref.shape[0]

    ks = []
    for s in range(B):
        x = l1_ref[s] - l0_ref[s]
        fg = tgt_ref[s] == 1
        mkey = jnp.where(fg, _key_of(x), jnp.int32(_INT_MAX))
        mkey_ref[s] = mkey
        h16_ref[s] = (mkey >> 16).astype(jnp.int16)
        n = jnp.sum(fg.astype(jnp.int32))
        ks.append(jnp.maximum(
            jnp.int32(1),
            jnp.floor(n.astype(jnp.float32) * jnp.float32(_K_FRAC)).astype(jnp.int32),
        ))

    buckets, cbls = _search16(lambda s: h16_ref[s], ks, B, -(2**15), 2**15 - 1)

    for s in range(B):
        mkey = mkey_ref[s]
        hk = mkey >> 16
        low = (mkey & jnp.int32(0xFFFF)) - jnp.int32(32768)
        key2 = jnp.where(
            hk == buckets[s], low,
            jnp.where(hk < buckets[s], jnp.int32(-32768), jnp.int32(32767)),
        )
        h16_ref[s] = key2.astype(jnp.int16)

    lows, _ = _search16(lambda s: h16_ref[s], ks, B, -(2**15), 2**15 - 1)

    acc = jnp.float32(0.0)
    for s in range(B):
        thr = (buckets[s] << 16) | (lows[s] + jnp.int32(32768))
        thr_f = jax.lax.bitcast_convert_type(
            jnp.where(thr < 0, thr ^ jnp.int32(0x7FFFFFFF), thr), jnp.float32)
        x = l1_ref[s] - l0_ref[s]
        fg = tgt_ref[s] == 1
        p = jax.nn.sigmoid(x)
        le = x <= thr_f
        keep = jnp.logical_and(fg, le)
        s_all = jnp.sum(p)
        pf = jnp.where(fg, p, jnp.float32(0.0))
        s_fg = jnp.sum(pf)
        s_keep = jnp.sum(jnp.where(le, pf, jnp.float32(0.0)))
        c_keep = jnp.sum(keep.astype(jnp.int32))
        union = s_all - s_fg + s_keep + c_keep.astype(jnp.float32)
        acc += ((2.0 * s_keep + jnp.float32(_SMOOTH))
                / (union + jnp.float32(_SMOOTH)))

    out_ref[0] = jnp.float32(1.0) - acc / jnp.float32(B)


def kernel(logits, target):
    logits = logits.astype(jnp.float32)
    B, _, H, W = logits.shape
    R = (H * W) // 512
    l0 = logits[:, 0].reshape(B, R, 512)
    l1 = logits[:, 1].reshape(B, R, 512)
    tgt = target.astype(jnp.int32).reshape(B, R, 512)

    out = pl.pallas_call(
        _dice_body,
        in_specs=[
            pl.BlockSpec(l0.shape, lambda: (0, 0, 0)),
            pl.BlockSpec(l1.shape, lambda: (0, 0, 0)),
            pl.BlockSpec(tgt.shape, lambda: (0, 0, 0)),
        ],
        out_specs=pl.BlockSpec(memory_space=pltpu.SMEM),
        out_shape=jax.ShapeDtypeStruct((1,), jnp.float32),
        scratch_shapes=[
            pltpu.VMEM((B, R, 512), jnp.int32),
            pltpu.VMEM((B, R, 512), jnp.int16),
        ],
    )(l0, l1, tgt)
    return out[0]


# speedup vs baseline: 1.6548x; 1.0072x over previous
import jax
import jax.numpy as jnp
from jax.experimental import pallas as pl
from jax.experimental.pallas import tpu as pltpu

_SMOOTH = 1e-05
_K_FRAC = 10.0 / 100.0
_INT_MAX = 2**31 - 1


def _key_of(x):
    ki = jax.lax.bitcast_convert_type(x, jnp.int32)
    return jnp.where(ki < 0, ki ^ jnp.int32(0x7FFFFFFF), ki)


def _count_le(plane_i16, mid_i32):
    t = (plane_i16 <= mid_i32.astype(jnp.int16)).astype(jnp.int16)
    while t.shape[0] > 16:
        h = t.shape[0] // 2
        t = t[:h] + t[h:]
    return jnp.sum(t.astype(jnp.int32))


def _search16(plane_refs, ks, B, lo0, hi0):
    def body(_, carry):
        los, his, cbls = carry
        nlo, nhi, ncb = [], [], []
        for s in range(B):
            lo, hi, cbl = los[s], his[s], cbls[s]
            mid = (lo + hi) >> 1
            c = _count_le(plane_refs(s), mid)
            pred = c >= ks[s]
            nlo.append(jnp.where(pred, lo, mid + 1))
            nhi.append(jnp.where(pred, mid, hi))
            ncb.append(jnp.where(pred, cbl, c))
        return (tuple(nlo), tuple(nhi), tuple(ncb))

    init = (
        tuple(jnp.int32(lo0) for _ in range(B)),
        tuple(jnp.int32(hi0) for _ in range(B)),
        tuple(jnp.int32(0) for _ in range(B)),
    )
    _, thrs, cbls = jax.lax.fori_loop(0, 16, body, init, unroll=4)
    return thrs, cbls


def _dice_body(l0_ref, l1_ref, tgt_ref, out_ref, mkey_ref, h16_ref):
    B = l0_ref.shape[0]

    ks = []
    for s in range(B):
        x = l1_ref[s] - l0_ref[s]
        fg = tgt_ref[s] == 1
        mkey = jnp.where(fg, _key_of(x), jnp.int32(_INT_MAX))
        mkey_ref[s] = mkey
        h16_ref[s] = (mkey >> 16).astype(jnp.int16)
        n = jnp.sum(fg.astype(jnp.int32))
        ks.append(jnp.maximum(
            jnp.int32(1),
            jnp.floor(n.astype(jnp.float32) * jnp.float32(_K_FRAC)).astype(jnp.int32),
        ))

    buckets, cbls = _search16(lambda s: h16_ref[s], ks, B, -(2**15), 2**15 - 1)

    for s in range(B):
        mkey = mkey_ref[s]
        hk = mkey >> 16
        low = (mkey & jnp.int32(0xFFFF)) - jnp.int32(32768)
        key2 = jnp.where(
            hk == buckets[s], low,
            jnp.where(hk < buckets[s], jnp.int32(-32768), jnp.int32(32767)),
        )
        h16_ref[s] = key2.astype(jnp.int16)

    lows, _ = _search16(lambda s: h16_ref[s], ks, B, -(2**15), 2**15 - 1)

    acc = jnp.float32(0.0)
    for s in range(B):
        thr = (buckets[s] << 16) | (lows[s] + jnp.int32(32768))
        thr_f = jax.lax.bitcast_convert_type(
            jnp.where(thr < 0, thr ^ jnp.int32(0x7FFFFFFF), thr), jnp.float32)
        x = l1_ref[s] - l0_ref[s]
        fg = tgt_ref[s] == 1
        p = jax.nn.sigmoid(x)
        le = x <= thr_f
        keep = jnp.logical_and(fg, le)
        s_all = jnp.sum(p)
        pf = jnp.where(fg, p, jnp.float32(0.0))
        s_fg = jnp.sum(pf)
        s_keep = jnp.sum(jnp.where(le, pf, jnp.float32(0.0)))
        c_keep = jnp.sum(keep.astype(jnp.int32))
        union = s_all - s_fg + s_keep + c_keep.astype(jnp.float32)
        acc += ((2.0 * s_keep + jnp.float32(_SMOOTH))
                / (union + jnp.float32(_SMOOTH)))

    out_ref[0] = jnp.float32(1.0) - acc / jnp.float32(B)


def kernel(logits, target):
    logits = logits.astype(jnp.float32)
    B, _, H, W = logits.shape
    R = (H * W) // 512
    l0 = logits[:, 0].reshape(B, R, 512)
    l1 = logits[:, 1].reshape(B, R, 512)
    tgt = target.astype(jnp.int32).reshape(B, R, 512)

    out = pl.pallas_call(
        _dice_body,
        in_specs=[
            pl.BlockSpec(l0.shape, lambda: (0, 0, 0)),
            pl.BlockSpec(l1.shape, lambda: (0, 0, 0)),
            pl.BlockSpec(tgt.shape, lambda: (0, 0, 0)),
        ],
        out_specs=pl.BlockSpec(memory_space=pltpu.SMEM),
        out_shape=jax.ShapeDtypeStruct((1,), jnp.float32),
        scratch_shapes=[
            pltpu.VMEM((B, R, 512), jnp.int32),
            pltpu.VMEM((B, R, 512), jnp.int16),
        ],
    )(l0, l1, tgt)
    return out[0]
